# baseline (speedup 1.0000x reference)
import jax
import jax.numpy as jnp
from jax.experimental import pallas as pl

_DIM = 256
_NCH = 2
_NEIGHBORS = 50
_N = 512
_B = 4
_SCALE = _DIM ** (-0.5)


def _edge_kernel(x_ref, wq0_ref, wk0_ref, wq1_ref, wk1_ref, out_ref):
    x = x_ref[...].reshape(_B * _N, _DIM)
    attns = []
    for wq_ref, wk_ref in ((wq0_ref, wk0_ref), (wq1_ref, wk1_ref)):
        q = jnp.dot(x, wq_ref[...], preferred_element_type=jnp.float32)
        k = jnp.dot(x, wk_ref[...], preferred_element_type=jnp.float32)
        logits = jnp.concatenate([
            jax.lax.dot_general(
                q[b * _N:(b + 1) * _N], k[b * _N:(b + 1) * _N],
                (((1,), (1,)), ((), ())),
                preferred_element_type=jnp.float32)
            for b in range(_B)
        ], axis=0) * _SCALE
        m = jnp.max(logits, axis=-1, keepdims=True)
        e = jnp.exp(logits - m)
        s = jnp.sum(e, axis=-1, keepdims=True)
        attns.append(e / s)

    sum_edge = attns[0] + attns[1]
    bits = jax.lax.bitcast_convert_type(sum_edge, jnp.int32)
    rowmax = jnp.max(bits, axis=-1, keepdims=True)

    ones_col = jnp.ones((128, 128), jnp.float32)

    def body(carry):
        lo, hi = carry
        mid = lo + jax.lax.shift_right_logical(hi - lo, 1)
        ge = (bits >= mid).astype(jnp.float32)
        folded = (ge[:, 0:128] + ge[:, 128:256]) + (ge[:, 256:384] + ge[:, 384:512])
        cnt = jnp.dot(folded, ones_col,
                      preferred_element_type=jnp.float32)[:, :1]
        pred = cnt >= float(_NEIGHBORS)
        return jnp.where(pred, mid, lo), jnp.where(pred, hi, mid)

    keys = sum_edge.astype(jnp.bfloat16)
    ones_col16 = jnp.ones((128, 128), jnp.bfloat16)

    def body16(carry):
        lo, hi = carry
        mid = lo + jax.lax.shift_right_logical(hi - lo, 1)
        mid_bf = jax.lax.bitcast_convert_type(
            jax.lax.shift_left(mid, 16), jnp.float32).astype(jnp.bfloat16)
        ge = (keys >= mid_bf).astype(jnp.bfloat16)
        folded = (ge[:, 0:128] + ge[:, 128:256]) + (ge[:, 256:384] + ge[:, 384:512])
        cnt = jnp.dot(folded, ones_col16,
                      preferred_element_type=jnp.float32)[:, :1]
        pred = cnt >= float(_NEIGHBORS)
        return jnp.where(pred, mid, lo), jnp.where(pred, hi, mid)

    zeros = jnp.zeros_like(rowmax)
    carry16 = (zeros, jax.lax.shift_right_logical(rowmax, 16) + 2)
    for _ in range(16):
        carry16 = body16(carry16)
    lo16 = carry16[0]

    center = jax.lax.shift_left(lo16, 16)
    carry = (jnp.maximum(center - 0x8001, 0), center + 0x8001)
    for _ in range(17):
        carry = body(carry)
    v50 = carry[0]
    v51 = jnp.max(jnp.where(bits < v50, bits, 0), axis=-1, keepdims=True)
    thr = v51 + jax.lax.shift_right_logical(v50 - v51 + 1, 1)

    row_ids = jax.lax.broadcasted_iota(jnp.int32, (_B * _N, _N), 0)
    col_ids = jax.lax.broadcasted_iota(jnp.int32, (_B * _N, _N), 1)
    diag = (row_ids % _N) == col_ids
    mask = (bits >= thr) | diag

    for c in range(_NCH):
        edge = jnp.where(mask, attns[c], 0.0)
        nr = edge / (jnp.sum(edge, axis=-1, keepdims=True) + 1e-6)
        for b in range(_B):
            nr_b = nr[b * _N:(b + 1) * _N]
            nc_b = nr_b / (jnp.sum(nr_b, axis=0, keepdims=True) + 1e-6)
            out_ref[b, c] = jax.lax.dot_general(
                nr_b, nc_b, (((1,), (1,)), ((), ())),
                preferred_element_type=jnp.float32)


def kernel(x, W):
    B, N, D = x.shape
    Wq0 = W[0 * D:1 * D].T
    Wq1 = W[1 * D:2 * D].T
    Wk0 = W[2 * D:3 * D].T
    Wk1 = W[3 * D:4 * D].T
    return pl.pallas_call(
        _edge_kernel,
        in_specs=[
            pl.BlockSpec((B, N, D), lambda: (0, 0, 0)),
            pl.BlockSpec((D, D), lambda: (0, 0)),
            pl.BlockSpec((D, D), lambda: (0, 0)),
            pl.BlockSpec((D, D), lambda: (0, 0)),
            pl.BlockSpec((D, D), lambda: (0, 0)),
        ],
        out_specs=pl.BlockSpec((B, _NCH, N, N), lambda: (0, 0, 0, 0)),
        out_shape=jax.ShapeDtypeStruct((B, _NCH, N, N), jnp.float32),
    )(x, Wq0, Wk0, Wq1, Wk1)


# speedup vs baseline: 1.5229x; 1.5229x over previous
import jax
import jax.numpy as jnp
from jax.experimental import pallas as pl

_DIM = 256
_NCH = 2
_NEIGHBORS = 50
_N = 512
_B = 4
_SCALE = _DIM ** (-0.5)


def _edge_kernel(x_ref, wq0_ref, wk0_ref, wq1_ref, wk1_ref, out_ref):
    x = x_ref[...].reshape(_B * _N, _DIM)
    attns = []
    for wq_ref, wk_ref in ((wq0_ref, wk0_ref), (wq1_ref, wk1_ref)):
        q = jnp.dot(x, wq_ref[...], preferred_element_type=jnp.float32)
        k = jnp.dot(x, wk_ref[...], preferred_element_type=jnp.float32)
        logits = jnp.concatenate([
            jax.lax.dot_general(
                q[b * _N:(b + 1) * _N], k[b * _N:(b + 1) * _N],
                (((1,), (1,)), ((), ())),
                preferred_element_type=jnp.float32)
            for b in range(_B)
        ], axis=0) * _SCALE
        m = jnp.max(logits, axis=-1, keepdims=True)
        e = jnp.exp(logits - m)
        s = jnp.sum(e, axis=-1, keepdims=True)
        attns.append(e / s)

    sum_edge = attns[0] + attns[1]
    bits = jax.lax.bitcast_convert_type(sum_edge, jnp.int32)
    bits_t = jnp.transpose(bits)

    lo = jnp.zeros((1, _B * _N), jnp.int32)
    for i in range(30, -1, -1):
        t = lo + (1 << i)
        ge = (bits_t >= t).astype(jnp.float32)
        cnt = jnp.sum(ge, axis=0, keepdims=True)
        lo = jnp.where(cnt >= float(_NEIGHBORS), t, lo)
    v50_t = lo
    v51_t = jnp.max(jnp.where(bits_t < v50_t, bits_t, 0), axis=0, keepdims=True)
    thr_t = v51_t + jax.lax.shift_right_logical(v50_t - v51_t + 1, 1)
    thr = jnp.transpose(thr_t)

    row_ids = jax.lax.broadcasted_iota(jnp.int32, (_B * _N, _N), 0)
    col_ids = jax.lax.broadcasted_iota(jnp.int32, (_B * _N, _N), 1)
    diag = (row_ids % _N) == col_ids
    mask = (bits >= thr) | diag

    for c in range(_NCH):
        edge = jnp.where(mask, attns[c], 0.0)
        nr = edge / (jnp.sum(edge, axis=-1, keepdims=True) + 1e-6)
        for b in range(_B):
            nr_b = nr[b * _N:(b + 1) * _N]
            nc_b = nr_b / (jnp.sum(nr_b, axis=0, keepdims=True) + 1e-6)
            out_ref[b, c] = jax.lax.dot_general(
                nr_b, nc_b, (((1,), (1,)), ((), ())),
                preferred_element_type=jnp.float32)


def kernel(x, W):
    B, N, D = x.shape
    Wq0 = W[0 * D:1 * D].T
    Wq1 = W[1 * D:2 * D].T
    Wk0 = W[2 * D:3 * D].T
    Wk1 = W[3 * D:4 * D].T
    return pl.pallas_call(
        _edge_kernel,
        in_specs=[
            pl.BlockSpec((B, N, D), lambda: (0, 0, 0)),
            pl.BlockSpec((D, D), lambda: (0, 0)),
            pl.BlockSpec((D, D), lambda: (0, 0)),
            pl.BlockSpec((D, D), lambda: (0, 0)),
            pl.BlockSpec((D, D), lambda: (0, 0)),
        ],
        out_specs=pl.BlockSpec((B, _NCH, N, N), lambda: (0, 0, 0, 0)),
        out_shape=jax.ShapeDtypeStruct((B, _NCH, N, N), jnp.float32),
    )(x, Wq0, Wk0, Wq1, Wk1)
